# baseline (speedup 1.0000x reference)
import jax
import jax.numpy as jnp
from jax import lax
from jax.experimental import pallas as pl
from jax.experimental.pallas import tpu as pltpu
from jax.experimental.pallas import tpu_sc as plsc

B = 4096
DENSE = 13
NSPARSE = 26
HIST = 50
VOCAB = 100000
ED = 32
OUT_D = DENSE + NSPARSE * ED + ED

NW = 32
ROWS_PER_W = B // NW
CHUNK = 32
NCHUNK = ROWS_PER_W // CHUNK


def _sc_body(sidx_hbm, vidx_hbm, stab_hbm, vtab_hbm,
             osparse_hbm, ovarlen_hbm,
             sidx_v, vidx_v, srows_v, vrows_v, pooled_v,
             sem_s, sem_v):
  wid = lax.axis_index("c") * 16 + lax.axis_index("s")

  def chunk_body(i, carry):
    gchunk = wid * NCHUNK + i
    rows = gchunk * CHUNK
    pltpu.sync_copy(sidx_hbm.at[pl.ds(gchunk * CHUNK * NSPARSE,
                                      CHUNK * NSPARSE)], sidx_v)
    pltpu.sync_copy(vidx_hbm.at[pl.ds(gchunk * CHUNK * HIST,
                                      CHUNK * HIST)], vidx_v)
    cp_s = pltpu.async_copy(stab_hbm.at[sidx_v], srows_v, sem_s)
    cp_v = pltpu.async_copy(vtab_hbm.at[vidx_v], vrows_v, sem_v)
    cp_v.wait()
    def row_body(b, c2):
      for half in range(2):
        def hbody(h, acc):
          return acc + vrows_v[b * HIST + h, pl.ds(half * 16, 16)]
        acc = lax.fori_loop(0, HIST, hbody, jnp.zeros((16,), jnp.float32))
        pooled_v[b, pl.ds(half * 16, 16)] = acc * (1.0 / HIST)
      return c2
    lax.fori_loop(0, CHUNK, row_body, 0)
    pltpu.sync_copy(pooled_v, ovarlen_hbm.at[pl.ds(rows, CHUNK)])
    cp_s.wait()
    pltpu.sync_copy(srows_v,
                    osparse_hbm.at[pl.ds(gchunk * CHUNK * NSPARSE,
                                         CHUNK * NSPARSE)])
    return carry

  lax.fori_loop(0, NCHUNK, chunk_body, 0)


@jax.jit
def kernel(x, sparse_tables, varlen_table):
  sidx_flat = (x[:, DENSE:DENSE + NSPARSE].astype(jnp.int32)
               + (jnp.arange(NSPARSE, dtype=jnp.int32) * VOCAB)[None, :]
               ).reshape(-1)
  vidx_flat = x[:, DENSE + NSPARSE:].astype(jnp.int32).reshape(-1)
  stab = sparse_tables.reshape(NSPARSE * VOCAB, ED)

  run = pl.kernel(
      _sc_body,
      out_type=(
          jax.ShapeDtypeStruct((B * NSPARSE, ED), jnp.float32),
          jax.ShapeDtypeStruct((B, ED), jnp.float32),
      ),
      mesh=plsc.VectorSubcoreMesh(core_axis_name="c", subcore_axis_name="s"),
      compiler_params=pltpu.CompilerParams(use_tc_tiling_on_sc=False),
      scratch_types=[
          pltpu.VMEM((CHUNK * NSPARSE,), jnp.int32),
          pltpu.VMEM((CHUNK * HIST,), jnp.int32),
          pltpu.VMEM((CHUNK * NSPARSE, ED), jnp.float32),
          pltpu.VMEM((CHUNK * HIST, ED), jnp.float32),
          pltpu.VMEM((CHUNK, ED), jnp.float32),
          pltpu.SemaphoreType.DMA,
          pltpu.SemaphoreType.DMA,
      ],
  )
  osparse, ovarlen = run(sidx_flat, vidx_flat, stab, varlen_table)
  return jnp.concatenate(
      [x[:, :DENSE], osparse.reshape(B, NSPARSE * ED), ovarlen], axis=1)


# speedup vs baseline: 1.6214x; 1.6214x over previous
import jax
import jax.numpy as jnp
from jax import lax
from jax.experimental import pallas as pl
from jax.experimental.pallas import tpu as pltpu
from jax.experimental.pallas import tpu_sc as plsc

B = 4096
DENSE = 13
NSPARSE = 26
HIST = 50
VOCAB = 100000
ED = 32
OUT_D = DENSE + NSPARSE * ED + ED

VCHUNK = 256
NGRP = B // 16


def _sc_body(sidxT_hbm, vidxT_hbm, denseT_hbm, stT_hbm, vtT_hbm, outT_hbm,
             plane_v, idx_v, vchunk_v, col_v, sem):
  wid = lax.axis_index("c") * 16 + lax.axis_index("s")

  @pl.when(wid < DENSE)
  def _():
    pltpu.sync_copy(denseT_hbm.at[wid], col_v)
    pltpu.sync_copy(col_v, outT_hbm.at[wid])

  pltpu.sync_copy(vtT_hbm.at[wid], plane_v)

  def vchunk_body(c, carry):
    pltpu.sync_copy(vidxT_hbm.at[:, pl.ds(c * VCHUNK, VCHUNK)], vchunk_v)

    def vgrp_body(g, carry2):
      def hbody(h, acc):
        idx = vchunk_v[h, pl.ds(g * 16, 16)]
        return acc + plsc.load_gather(plane_v, [idx])
      acc = lax.fori_loop(0, HIST, hbody, jnp.zeros((16,), jnp.float32))
      col_v[pl.ds(c * VCHUNK + g * 16, 16)] = acc * (1.0 / HIST)
      return carry2

    lax.fori_loop(0, VCHUNK // 16, vgrp_body, 0)
    return carry

  lax.fori_loop(0, B // VCHUNK, vchunk_body, 0)
  pltpu.sync_copy(col_v, outT_hbm.at[DENSE + NSPARSE * ED + wid])

  def fbody(f, carry):
    pltpu.sync_copy(stT_hbm.at[f, wid], plane_v)
    pltpu.sync_copy(sidxT_hbm.at[f], idx_v)

    def gbody(g, carry2):
      idx = idx_v[pl.ds(g * 16, 16)]
      col_v[pl.ds(g * 16, 16)] = plsc.load_gather(plane_v, [idx])
      return carry2

    lax.fori_loop(0, NGRP, gbody, 0)
    pltpu.sync_copy(col_v, outT_hbm.at[DENSE + f * ED + wid])
    return carry

  lax.fori_loop(0, NSPARSE, fbody, 0)


@jax.jit
def kernel(x, sparse_tables, varlen_table):
  xT = x.T
  sidxT = xT[DENSE:DENSE + NSPARSE].astype(jnp.int32)
  vidxT = xT[DENSE + NSPARSE:].astype(jnp.int32)
  denseT = xT[:DENSE]
  stT = sparse_tables.transpose(0, 2, 1)
  vtT = varlen_table.T

  run = pl.kernel(
      _sc_body,
      out_type=jax.ShapeDtypeStruct((OUT_D, B), jnp.float32),
      mesh=plsc.VectorSubcoreMesh(core_axis_name="c", subcore_axis_name="s"),
      compiler_params=pltpu.CompilerParams(
          use_tc_tiling_on_sc=False, needs_layout_passes=False),
      scratch_types=[
          pltpu.VMEM((VOCAB,), jnp.float32),
          pltpu.VMEM((B,), jnp.int32),
          pltpu.VMEM((HIST, VCHUNK), jnp.int32),
          pltpu.VMEM((B,), jnp.float32),
          pltpu.SemaphoreType.DMA,
      ],
  )
  outT = run(sidxT, vidxT, denseT, stT, vtT)
  return outT.T


# speedup vs baseline: 2.5085x; 1.5471x over previous
import jax
import jax.numpy as jnp
from jax import lax
from jax.experimental import pallas as pl
from jax.experimental.pallas import tpu as pltpu
from jax.experimental.pallas import tpu_sc as plsc

B = 4096
DENSE = 13
NSPARSE = 26
HIST = 50
VOCAB = 100000
ED = 32
OUT_D = DENSE + NSPARSE * ED + ED

VC = 5120
NCH = 19
TAIL0 = NCH * VC
TAILN = VOCAB - TAIL0
TAILP = 2944
VPLANE = 102400
NUNIT = NSPARSE * (ED // 8)


def _sc_body(sidx_hbm, vidx_hbm, dense_hbm, stT_hbm, tail_hbm, vt_hbm,
             offs_hbm, vs_hbm, bs_hbm, out_hbm, sem):
  wid = lax.axis_index("c") * 16 + lax.axis_index("s")

  def varlen_phase(plane_v, vch_v, col_v):
    @pl.when(wid < DENSE)
    def _():
      pltpu.sync_copy(dense_hbm.at[pl.ds(wid * B, B)], col_v)
      pltpu.sync_copy(col_v, out_hbm.at[pl.ds(wid * B, B)])

    pltpu.sync_copy(vt_hbm.at[pl.ds(wid * VPLANE, VPLANE)], plane_v)

    def vchunk_body(c, carry):
      pltpu.sync_copy(vidx_hbm.at[pl.ds(c * (HIST * 256), HIST * 256)],
                      vch_v)

      def vgrp_body(g, carry2):
        def hbody(h, acc):
          idx = vch_v[pl.ds(h * 256 + g * 16, 16)]
          return acc + plsc.load_gather(plane_v, [idx])
        acc = lax.fori_loop(0, HIST, hbody, jnp.zeros((16,), jnp.float32))
        col_v[pl.ds(c * 256 + g * 16, 16)] = acc * (1.0 / HIST)
        return carry2

      lax.fori_loop(0, 16, vgrp_body, 0)
      return carry

    lax.fori_loop(0, B // 256, vchunk_body, 0)
    pltpu.sync_copy(
        col_v, out_hbm.at[pl.ds((DENSE + NSPARSE * ED + wid) * B, B)])

  pl.run_scoped(
      varlen_phase,
      pltpu.VMEM((VPLANE,), jnp.float32),
      pltpu.VMEM((HIST * 256,), jnp.int32),
      pltpu.VMEM((B,), jnp.float32),
  )

  def sparse_phase(slab_v, tail_v, vs_v, bs_v, offs_v, *cols):
    def do_unit(u):
      f = lax.rem(u, NSPARSE)
      j = lax.div(u, NSPARSE)
      pltpu.sync_copy(vs_hbm.at[pl.ds(f * B, B)], vs_v.at[pl.ds(0, B)])
      pltpu.sync_copy(bs_hbm.at[pl.ds(f * B, B)], bs_v.at[pl.ds(0, B)])
      pltpu.sync_copy(offs_hbm.at[pl.ds(f * 40, 40)], offs_v.at[pl.ds(0, 40)])

      def chunk_body(c, carry):
        pltpu.sync_copy(
            stT_hbm.at[f, pl.ds(j * 8, 8), pl.ds(c * VC, VC)], slab_v)
        se = offs_v[pl.ds(c, 16)]
        start, end = se[0], se[1]

        def grp(g, carry2):
          pos = start + g * 16
          v16 = vs_v[pl.ds(pos, 16)]
          b16 = bs_v[pl.ds(pos, 16)]
          mask = (pos + lax.iota(jnp.int32, 16)) < end
          vrel = v16 - c * VC
          for e in range(8):
            vals = plsc.load_gather(
                slab_v, [jnp.full((16,), e, jnp.int32), vrel], mask=mask)
            plsc.store_scatter(cols[e], [b16], vals, mask=mask)
          return carry2

        lax.fori_loop(0, lax.div(end - start + 15, 16), grp, 0)
        return carry

      lax.fori_loop(0, NCH, chunk_body, 0)

      pltpu.sync_copy(
          tail_hbm.at[pl.ds((f * ED + j * 8) * TAILP, 8 * TAILP)], tail_v)
      se = offs_v[pl.ds(NCH, 16)]
      start, end = se[0], se[1]

      def tgrp(g, carry2):
        pos = start + g * 16
        v16 = vs_v[pl.ds(pos, 16)]
        b16 = bs_v[pl.ds(pos, 16)]
        mask = (pos + lax.iota(jnp.int32, 16)) < end
        vrel = v16 - TAIL0
        for e in range(8):
          vals = plsc.load_gather(
              tail_v, [vrel + e * TAILP], mask=mask)
          plsc.store_scatter(cols[e], [b16], vals, mask=mask)
        return carry2

      lax.fori_loop(0, lax.div(end - start + 15, 16), tgrp, 0)

      for e in range(8):
        pltpu.sync_copy(
            cols[e],
            out_hbm.at[pl.ds((DENSE + f * ED + j * 8 + e) * B, B)])

    def unit_loop(k, carry):
      u = wid + k * 32

      @pl.when(u < NUNIT)
      def _():
        do_unit(u)
      return carry

    lax.fori_loop(0, 4, unit_loop, 0)

  pl.run_scoped(
      sparse_phase,
      pltpu.VMEM((8, VC), jnp.float32),
      pltpu.VMEM((8 * TAILP,), jnp.float32),
      pltpu.VMEM((B + 32,), jnp.int32),
      pltpu.VMEM((B + 32,), jnp.int32),
      pltpu.VMEM((64,), jnp.int32),
      *[pltpu.VMEM((B,), jnp.float32) for _ in range(8)],
  )


@jax.jit
def kernel(x, sparse_tables, varlen_table):
  xT = x.T
  sidxT = xT[DENSE:DENSE + NSPARSE].astype(jnp.int32)
  vidx_ch = (xT[DENSE + NSPARSE:].astype(jnp.int32)
             .reshape(HIST, 16, 256).transpose(1, 0, 2).reshape(-1))
  dense_flat = xT[:DENSE].reshape(-1)
  stT = sparse_tables.transpose(0, 2, 1)
  vt_pad = jnp.pad(varlen_table.T,
                   ((0, 0), (0, VPLANE - VOCAB))).reshape(-1)
  tail_lin = jnp.pad(stT[:, :, TAIL0:],
                     ((0, 0), (0, 0), (0, TAILP - TAILN))).reshape(-1)

  order = jnp.argsort(sidxT, axis=1).astype(jnp.int32)
  v_sorted = jnp.take_along_axis(sidxT, order, axis=1)
  bounds = (jnp.arange(1, NCH + 1, dtype=jnp.int32) * VC)
  inner = jax.vmap(
      lambda vs: jnp.searchsorted(vs, bounds).astype(jnp.int32))(v_sorted)
  offs = jnp.concatenate(
      [jnp.zeros((NSPARSE, 1), jnp.int32), inner,
       jnp.full((NSPARSE, 1), B, jnp.int32),
       jnp.zeros((NSPARSE, 40 - (NCH + 2)), jnp.int32)], axis=1)

  run = pl.kernel(
      _sc_body,
      out_type=jax.ShapeDtypeStruct((OUT_D * B,), jnp.float32),
      mesh=plsc.VectorSubcoreMesh(core_axis_name="c", subcore_axis_name="s"),
      compiler_params=pltpu.CompilerParams(
          use_tc_tiling_on_sc=True, needs_layout_passes=False),
      scratch_types=[pltpu.SemaphoreType.DMA],
  )
  out_flat = run(sidxT.reshape(-1), vidx_ch, dense_flat, stT, tail_lin,
                 vt_pad, offs.reshape(-1), v_sorted.reshape(-1),
                 order.reshape(-1))
  return out_flat.reshape(OUT_D, B).T


# speedup vs baseline: 3.6963x; 1.4735x over previous
import jax
import jax.numpy as jnp
from jax import lax
from jax.experimental import pallas as pl
from jax.experimental.pallas import tpu as pltpu
from jax.experimental.pallas import tpu_sc as plsc

B = 4096
DENSE = 13
NSPARSE = 26
HIST = 50
VOCAB = 100000
ED = 32
OUT_D = DENSE + NSPARSE * ED + ED

VC = 3840
NCH = 25
TAIL0 = NCH * VC
TAILP = 4096
VPLANE = 102400
NUNIT = NSPARSE * (ED // 8)


def _sc_body(vidx_hbm, dense_hbm, stT_hbm, tail_hbm, vt_hbm,
             offs_hbm, vs_hbm, bs_hbm, out_hbm, sem):
  wid = lax.axis_index("c") * 16 + lax.axis_index("s")

  def varlen_phase(plane_v, vch_v, col_v):
    @pl.when(wid < DENSE)
    def _():
      pltpu.sync_copy(dense_hbm.at[pl.ds(wid * B, B)], col_v)
      pltpu.sync_copy(col_v, out_hbm.at[pl.ds(wid * B, B)])

    pltpu.sync_copy(vt_hbm.at[pl.ds(wid * VPLANE, VPLANE)], plane_v)

    def vchunk_body(c, carry):
      pltpu.sync_copy(vidx_hbm.at[pl.ds(c * (HIST * 256), HIST * 256)],
                      vch_v)

      def vgrp_body(g, carry2):
        def hbody(h, acc):
          idx = vch_v[pl.ds(h * 256 + g * 16, 16)]
          return acc + plsc.load_gather(plane_v, [idx])
        acc = lax.fori_loop(0, HIST, hbody, jnp.zeros((16,), jnp.float32),
                            unroll=5)
        col_v[pl.ds(c * 256 + g * 16, 16)] = acc * (1.0 / HIST)
        return carry2

      lax.fori_loop(0, 16, vgrp_body, 0)
      return carry

    lax.fori_loop(0, B // 256, vchunk_body, 0)
    pltpu.sync_copy(
        col_v, out_hbm.at[pl.ds((DENSE + NSPARSE * ED + wid) * B, B)])

  pl.run_scoped(
      varlen_phase,
      pltpu.VMEM((VPLANE,), jnp.float32),
      pltpu.VMEM((HIST * 256,), jnp.int32),
      pltpu.VMEM((B,), jnp.float32),
  )

  def sparse_phase(slab_a, slab_b, vs_v, bs_v, offs_v, sem_a, sem_b, *cols):
    def slab_cp(f, j, c, buf, s):
      return pltpu.make_async_copy(
          stT_hbm.at[f, pl.ds(j * 8, 8), pl.ds(c * VC, VC)],
          buf.at[:, pl.ds(0, VC)], s)

    def do_unit(u):
      f = lax.rem(u, NSPARSE)
      j = lax.div(u, NSPARSE)
      pltpu.sync_copy(vs_hbm.at[pl.ds(f * B, B)], vs_v.at[pl.ds(0, B)])
      pltpu.sync_copy(bs_hbm.at[pl.ds(f * B, B)], bs_v.at[pl.ds(0, B)])
      pltpu.sync_copy(offs_hbm.at[pl.ds(f * 40, 40)],
                      offs_v.at[pl.ds(0, 40)])

      def process(buf, c, base):
        se = offs_v[pl.ds(c, 16)]
        start, end = se[0], se[1]

        def grp(g, carry2):
          pos = start + g * 16
          v16 = vs_v[pl.ds(pos, 16)]
          b16 = bs_v[pl.ds(pos, 16)]
          mask = (pos + lax.iota(jnp.int32, 16)) < end
          vrel = v16 - base
          for e in range(8):
            vals = plsc.load_gather(
                buf, [jnp.full((16,), e, jnp.int32), vrel], mask=mask)
            plsc.store_scatter(cols[e], [b16], vals, mask=mask)
          return carry2

        lax.fori_loop(0, lax.div(end - start + 15, 16), grp, 0)

      slab_cp(f, j, 0, slab_a, sem_a).start()

      def pair(p, carry):
        c0 = 2 * p
        slab_cp(f, j, c0 + 1, slab_b, sem_b).start()
        slab_cp(f, j, c0, slab_a, sem_a).wait()
        process(slab_a, c0, c0 * VC)
        slab_cp(f, j, c0 + 2, slab_a, sem_a).start()
        slab_cp(f, j, c0 + 1, slab_b, sem_b).wait()
        process(slab_b, c0 + 1, (c0 + 1) * VC)
        return carry

      lax.fori_loop(0, (NCH - 1) // 2, pair, 0)
      slab_cp(f, j, NCH - 1, slab_a, sem_a).wait()
      process(slab_a, NCH - 1, (NCH - 1) * VC)
      pltpu.sync_copy(tail_hbm.at[f, pl.ds(j * 8, 8)], slab_a)
      process(slab_a, NCH, TAIL0)

      for e in range(8):
        pltpu.sync_copy(
            cols[e],
            out_hbm.at[pl.ds((DENSE + f * ED + j * 8 + e) * B, B)])

    def unit_loop(k, carry):
      do_unit(wid + k * 32)
      return carry

    lax.fori_loop(0, 3, unit_loop, 0)

    @pl.when(lax.rem(wid, 4) == 0)
    def _():
      do_unit(96 + lax.div(wid, 4))

  pl.run_scoped(
      sparse_phase,
      pltpu.VMEM((8, TAILP), jnp.float32),
      pltpu.VMEM((8, TAILP), jnp.float32),
      pltpu.VMEM((B + 32,), jnp.int32),
      pltpu.VMEM((B + 32,), jnp.int32),
      pltpu.VMEM((64,), jnp.int32),
      pltpu.SemaphoreType.DMA,
      pltpu.SemaphoreType.DMA,
      *[pltpu.VMEM((B,), jnp.float32) for _ in range(8)],
  )


@jax.jit
def kernel(x, sparse_tables, varlen_table):
  xT = x.T
  sidxT = xT[DENSE:DENSE + NSPARSE].astype(jnp.int32)
  vidx_ch = (xT[DENSE + NSPARSE:].astype(jnp.int32)
             .reshape(HIST, 16, 256).transpose(1, 0, 2).reshape(-1))
  dense_flat = xT[:DENSE].reshape(-1)
  stT = sparse_tables.transpose(0, 2, 1)
  vt_pad = jnp.pad(varlen_table.T,
                   ((0, 0), (0, VPLANE - VOCAB))).reshape(-1)
  tail3 = jnp.pad(stT[:, :, TAIL0:],
                  ((0, 0), (0, 0), (0, TAILP - (VOCAB - TAIL0))))

  binit = jnp.broadcast_to(jnp.arange(B, dtype=jnp.int32)[None, :],
                           (NSPARSE, B))
  v_sorted, b_sorted = lax.sort_key_val(sidxT, binit, dimension=1)
  bid = jnp.minimum(sidxT // VC, NCH)
  counts = jnp.sum(
      (bid[:, :, None] == jnp.arange(NCH + 1, dtype=jnp.int32)[None, None, :]
       ).astype(jnp.int32), axis=1)
  offs = jnp.concatenate(
      [jnp.zeros((NSPARSE, 1), jnp.int32),
       jnp.cumsum(counts, axis=1).astype(jnp.int32),
       jnp.zeros((NSPARSE, 40 - (NCH + 2)), jnp.int32)], axis=1)

  run = pl.kernel(
      _sc_body,
      out_type=jax.ShapeDtypeStruct((OUT_D * B,), jnp.float32),
      mesh=plsc.VectorSubcoreMesh(core_axis_name="c", subcore_axis_name="s"),
      compiler_params=pltpu.CompilerParams(
          use_tc_tiling_on_sc=True, needs_layout_passes=False),
      scratch_types=[pltpu.SemaphoreType.DMA],
  )
  out_flat = run(vidx_ch, dense_flat, stT, tail3, vt_pad,
                 offs.reshape(-1), v_sorted.reshape(-1),
                 b_sorted.reshape(-1))
  return out_flat.reshape(OUT_D, B).T


# speedup vs baseline: 3.9681x; 1.0735x over previous
import jax
import jax.numpy as jnp
from jax import lax
from jax.experimental import pallas as pl
from jax.experimental.pallas import tpu as pltpu
from jax.experimental.pallas import tpu_sc as plsc

B = 4096
DENSE = 13
NSPARSE = 26
HIST = 50
VOCAB = 100000
ED = 32
OUT_D = DENSE + NSPARSE * ED + ED

VC = 3840
NCH = 25
TAIL0 = NCH * VC
TAILW = 3968
LAST0 = TAIL0 + TAILW
TAILP = 4096
VPLANE = 102400
NUNIT = NSPARSE * (ED // 8)


def _sc_body(vidx_hbm, dense_hbm, stT_hbm, tail_hbm, vt_hbm,
             offs_hbm, vs_hbm, out_hbm, sem):
  wid = lax.axis_index("c") * 16 + lax.axis_index("s")

  def varlen_phase(plane_v, vch_v, col_v):
    @pl.when(wid < DENSE)
    def _():
      pltpu.sync_copy(dense_hbm.at[pl.ds(wid * B, B)], col_v)
      pltpu.sync_copy(col_v, out_hbm.at[pl.ds(wid * B, B)])

    pltpu.sync_copy(vt_hbm.at[pl.ds(wid * VPLANE, VPLANE)], plane_v)

    def vchunk_body(c, carry):
      pltpu.sync_copy(vidx_hbm.at[pl.ds(c * (HIST * 256), HIST * 256)],
                      vch_v)

      def vgrp_body(g, carry2):
        def hbody(h, acc):
          idx = vch_v[pl.ds(h * 256 + g * 16, 16)]
          return acc + plsc.load_gather(plane_v, [idx])
        acc = lax.fori_loop(0, HIST, hbody, jnp.zeros((16,), jnp.float32),
                            unroll=5)
        col_v[pl.ds(c * 256 + g * 16, 16)] = acc * (1.0 / HIST)
        return carry2

      lax.fori_loop(0, 16, vgrp_body, 0)
      return carry

    lax.fori_loop(0, B // 256, vchunk_body, 0)
    pltpu.sync_copy(
        col_v, out_hbm.at[pl.ds((DENSE + NSPARSE * ED + wid) * B, B)])

  pl.run_scoped(
      varlen_phase,
      pltpu.VMEM((VPLANE,), jnp.float32),
      pltpu.VMEM((HIST * 256,), jnp.int32),
      pltpu.VMEM((B,), jnp.float32),
  )

  def sparse_phase(slab_a, slab_b, vs_v, offs_v, sem_a, sem_b, *cols):
    def slab_cp(f, j, c, buf, s):
      return pltpu.make_async_copy(
          stT_hbm.at[f, pl.ds(j * 8, 8), pl.ds(c * VC, VC)],
          buf.at[:, pl.ds(0, VC)], s)

    def tail_cp(f, j, buf, s):
      return pltpu.make_async_copy(
          stT_hbm.at[f, pl.ds(j * 8, 8), pl.ds(TAIL0, TAILW)],
          buf.at[:, pl.ds(0, TAILW)], s)

    def last_cp(f, j, buf, s):
      return pltpu.make_async_copy(
          tail_hbm.at[f, pl.ds(j * 8, 8)], buf.at[:, pl.ds(0, 128)], s)

    def do_unit(u):
      f = lax.rem(u, NSPARSE)
      j = lax.div(u, NSPARSE)
      pltpu.sync_copy(vs_hbm.at[pl.ds(f * B, B)], vs_v.at[pl.ds(0, B)])
      pltpu.sync_copy(offs_hbm.at[pl.ds(f * 48, 48)],
                      offs_v.at[pl.ds(0, 48)])

      def process(buf, c, base):
        se = offs_v[pl.ds(c, 16)]
        start, end = se[0], se[1]

        def grp(g, carry2):
          pos = start + g * 16
          p16 = vs_v[pl.ds(pos, 16)]
          v16 = lax.shift_right_logical(p16, 12)
          b16 = lax.bitwise_and(p16, 4095)
          mask = (pos + lax.iota(jnp.int32, 16)) < end
          vrel = v16 - base
          for e in range(8):
            vals = plsc.load_gather(
                buf, [jnp.full((16,), e, jnp.int32), vrel], mask=mask)
            plsc.store_scatter(cols[e], [b16], vals, mask=mask)
          return carry2

        lax.fori_loop(0, lax.div(end - start + 15, 16), grp, 0)

      slab_cp(f, j, 0, slab_a, sem_a).start()

      def pair(p, carry):
        c0 = 2 * p
        slab_cp(f, j, c0 + 1, slab_b, sem_b).start()
        slab_cp(f, j, c0, slab_a, sem_a).wait()
        process(slab_a, c0, c0 * VC)
        slab_cp(f, j, c0 + 2, slab_a, sem_a).start()
        slab_cp(f, j, c0 + 1, slab_b, sem_b).wait()
        process(slab_b, c0 + 1, (c0 + 1) * VC)
        return carry

      lax.fori_loop(0, (NCH - 1) // 2, pair, 0)
      tail_cp(f, j, slab_b, sem_b).start()
      slab_cp(f, j, NCH - 1, slab_a, sem_a).wait()
      process(slab_a, NCH - 1, (NCH - 1) * VC)
      last_cp(f, j, slab_a, sem_a).start()
      tail_cp(f, j, slab_b, sem_b).wait()
      process(slab_b, NCH, TAIL0)
      last_cp(f, j, slab_a, sem_a).wait()
      process(slab_a, NCH + 1, LAST0)

      for e in range(8):
        pltpu.sync_copy(
            cols[e],
            out_hbm.at[pl.ds((DENSE + f * ED + j * 8 + e) * B, B)])

    def unit_loop(k, carry):
      do_unit(wid + k * 32)
      return carry

    lax.fori_loop(0, 3, unit_loop, 0)

    @pl.when(lax.rem(wid, 4) == 0)
    def _():
      do_unit(96 + lax.div(wid, 4))

  pl.run_scoped(
      sparse_phase,
      pltpu.VMEM((8, TAILP), jnp.float32),
      pltpu.VMEM((8, TAILP), jnp.float32),
      pltpu.VMEM((B + 32,), jnp.int32),
      pltpu.VMEM((64,), jnp.int32),
      pltpu.SemaphoreType.DMA,
      pltpu.SemaphoreType.DMA,
      *[pltpu.VMEM((B,), jnp.float32) for _ in range(8)],
  )


@jax.jit
def kernel(x, sparse_tables, varlen_table):
  xT = x.T
  sidxT = xT[DENSE:DENSE + NSPARSE].astype(jnp.int32)
  vidx_ch = (xT[DENSE + NSPARSE:].astype(jnp.int32)
             .reshape(HIST, 16, 256).transpose(1, 0, 2).reshape(-1))
  dense_flat = xT[:DENSE].reshape(-1)
  stT = sparse_tables.transpose(0, 2, 1)
  vt_pad = jnp.pad(varlen_table.T,
                   ((0, 0), (0, VPLANE - VOCAB))).reshape(-1)
  tail_aux = jnp.pad(stT[:, :, LAST0:],
                     ((0, 0), (0, 0), (0, 128 - (VOCAB - LAST0))))

  binit = jnp.broadcast_to(jnp.arange(B, dtype=jnp.int32)[None, :],
                           (NSPARSE, B))
  keys = jnp.sort(sidxT * B + binit, axis=1)
  bid = jnp.where(sidxT < TAIL0, sidxT // VC,
                  jnp.where(sidxT < LAST0, NCH, NCH + 1))
  counts = jnp.sum(
      (bid[:, :, None] == jnp.arange(NCH + 2, dtype=jnp.int32)[None, None, :]
       ).astype(jnp.int32), axis=1)
  offs = jnp.concatenate(
      [jnp.zeros((NSPARSE, 1), jnp.int32),
       jnp.cumsum(counts, axis=1).astype(jnp.int32),
       jnp.zeros((NSPARSE, 48 - (NCH + 3)), jnp.int32)], axis=1)

  run = pl.kernel(
      _sc_body,
      out_type=jax.ShapeDtypeStruct((OUT_D * B,), jnp.float32),
      mesh=plsc.VectorSubcoreMesh(core_axis_name="c", subcore_axis_name="s"),
      compiler_params=pltpu.CompilerParams(
          use_tc_tiling_on_sc=True, needs_layout_passes=False),
      scratch_types=[pltpu.SemaphoreType.DMA],
  )
  out_flat = run(vidx_ch, dense_flat, stT, tail_aux, vt_pad,
                 offs.reshape(-1), keys.reshape(-1))
  return out_flat.reshape(OUT_D, B).T
